# Initial kernel scaffold; baseline (speedup 1.0000x reference)
import functools

import jax
import jax.numpy as jnp
from jax import lax
from jax.experimental import pallas as pl
from jax.experimental.pallas import tpu as pltpu
from jax.experimental.pallas import tpu_sc as plsc

N_NODES = 10000
D = 128
E_EDGES = 320000
CE = 80
NCHUNKS = E_EDGES // CE
NC = 2
NS = 16
NW = NC * NS
CPW = NCHUNKS // NW
RPT = N_NODES // NS
W1 = 144
W2 = 128

_DOT = (((1,), (1,)), ((), ()))


def _make_seg_sum(width):
  mesh = plsc.VectorSubcoreMesh(core_axis_name="c", subcore_axis_name="s")

  @functools.partial(
      pl.kernel,
      out_type=jax.ShapeDtypeStruct((NC, N_NODES, width), jnp.float32),
      mesh=mesh,
      scratch_types=[
          pltpu.VMEM((CPW, CE), jnp.int32),
          pltpu.VMEM((CPW, CE), jnp.int32),
          pltpu.VMEM((CE,), jnp.int32),
          pltpu.VMEM((CE, width), jnp.float32),
          pltpu.VMEM_SHARED((N_NODES, width), jnp.float32),
      ],
  )
  def seg_sum(y_hbm, src_hbm, dst_hbm, zero_hbm, out_hbm,
              sidx, didx, dv, rows, acc):
    c = lax.axis_index("c")
    s = lax.axis_index("s")
    wid = s * NC + c
    pltpu.sync_copy(zero_hbm, acc.at[pl.ds(s * RPT, RPT)])
    pltpu.sync_copy(src_hbm.at[pl.ds(wid * CPW, CPW)], sidx)
    pltpu.sync_copy(dst_hbm.at[pl.ds(wid * CPW, CPW)], didx)
    plsc.subcore_barrier()

    @pl.loop(0, CPW)
    def _(j):
      for k16 in range(CE // 16):
        dv[pl.ds(k16 * 16, 16)] = didx[j, pl.ds(k16 * 16, 16)]
      pltpu.sync_copy(y_hbm.at[sidx.at[j]], rows)
      pltpu.sync_copy(rows, acc.at[dv], add=True)

    plsc.subcore_barrier()
    pltpu.sync_copy(acc.at[pl.ds(s * RPT, RPT)],
                    out_hbm.at[c, pl.ds(s * RPT, RPT)])

  return seg_sum


_seg_sum_w1 = _make_seg_sum(W1)
_seg_sum_w2 = _make_seg_sum(W2)


_BLK = 2000
_GRID = N_NODES // _BLK


def _pre_body(x_ref, wl_ref, wr_ref, b_ref, y_ref, z_ref):
  xb = x_ref[...]
  y_ref[:, :D] = lax.dot_general(xb, wl_ref[...], _DOT,
                                 preferred_element_type=jnp.float32)
  pat = (lax.broadcasted_iota(jnp.int32, (_BLK, W1 - D), 1) == 0)
  y_ref[:, D:] = pat.astype(jnp.float32)
  z_ref[...] = lax.dot_general(xb, wr_ref[...], _DOT,
                               preferred_element_type=jnp.float32) + b_ref[...]


def _mid_body(p_ref, z1_ref, wl_ref, wr_ref, b_ref, y2_ref, z2_ref, inv_ref):
  sacc = p_ref[0] + p_ref[1]
  cnt = sacc[:, D:D + 1]
  inv = 1.0 / jnp.maximum(cnt, 1.0)
  h = jnp.maximum(sacc[:, :D] * inv + z1_ref[...], 0.0)
  y2_ref[...] = lax.dot_general(h, wl_ref[...], _DOT,
                                preferred_element_type=jnp.float32)
  z2_ref[...] = lax.dot_general(h, wr_ref[...], _DOT,
                                preferred_element_type=jnp.float32) + b_ref[...]
  inv_ref[...] = jnp.broadcast_to(inv, (_BLK, D))


def _post_body(p_ref, z2_ref, inv_ref, o_ref):
  sacc = p_ref[0] + p_ref[1]
  o_ref[...] = sacc * inv_ref[...] + z2_ref[...]


def _row_spec(width):
  return pl.BlockSpec((_BLK, width), lambda i: (i, 0))


def _full_spec(shape):
  nd = len(shape)
  return pl.BlockSpec(shape, lambda i, _nd=nd: (0,) * _nd)


def _part_spec(width):
  return pl.BlockSpec((NC, _BLK, width), lambda i: (0, i, 0))


_pre = pl.pallas_call(
    _pre_body,
    grid=(_GRID,),
    in_specs=[_row_spec(D), _full_spec((D, D)), _full_spec((D, D)),
              _full_spec((1, D))],
    out_specs=[_row_spec(W1), _row_spec(D)],
    out_shape=[jax.ShapeDtypeStruct((N_NODES, W1), jnp.float32),
               jax.ShapeDtypeStruct((N_NODES, D), jnp.float32)],
)

_mid = pl.pallas_call(
    _mid_body,
    grid=(_GRID,),
    in_specs=[_part_spec(W1), _row_spec(D), _full_spec((D, D)),
              _full_spec((D, D)), _full_spec((1, D))],
    out_specs=[_row_spec(D), _row_spec(D), _row_spec(D)],
    out_shape=[jax.ShapeDtypeStruct((N_NODES, D), jnp.float32),
               jax.ShapeDtypeStruct((N_NODES, D), jnp.float32),
               jax.ShapeDtypeStruct((N_NODES, D), jnp.float32)],
)

_post = pl.pallas_call(
    _post_body,
    grid=(_GRID,),
    in_specs=[_part_spec(W2), _row_spec(D), _row_spec(D)],
    out_specs=_row_spec(D),
    out_shape=jax.ShapeDtypeStruct((N_NODES, D), jnp.float32),
)


def kernel(x, edge_index, W1l, b1l, W1r, W2l, b2l, W2r):
  src2 = edge_index[0].reshape(NCHUNKS, CE)
  dst2 = edge_index[1].reshape(NCHUNKS, CE)
  zero1 = jnp.zeros((RPT, W1), jnp.float32)
  zero2 = jnp.zeros((RPT, W2), jnp.float32)

  y1, z1 = _pre(x, W1l, W1r, b1l.reshape(1, D))
  p1 = _seg_sum_w1(y1, src2, dst2, zero1)
  y2, z2, inv = _mid(p1, z1, W2l, W2r, b2l.reshape(1, D))
  p2 = _seg_sum_w2(y2, src2, dst2, zero2)
  return _post(p2, z2, inv)


# speedup vs baseline: 6.8463x; 6.8463x over previous
import functools

import jax
import jax.numpy as jnp
from jax import lax
from jax.experimental import pallas as pl
from jax.experimental.pallas import tpu as pltpu
from jax.experimental.pallas import tpu_sc as plsc

N_NODES = 10000
D = 128
E_EDGES = 320000
CE = 80
NCHUNKS = E_EDGES // CE
NC = 2
NS = 16
NW = NC * NS
CPW = NCHUNKS // NW
N_PAD = 10240
RPT = N_PAD // NS

_DOT = (((1,), (1,)), ((), ()))


def _make_seg_sum(with_counts):
  mesh = plsc.VectorSubcoreMesh(core_axis_name="c", subcore_axis_name="s")
  out_type = [jax.ShapeDtypeStruct((NC, N_PAD, D), jnp.float32)]
  if with_counts:
    out_type = out_type + [jax.ShapeDtypeStruct((NC, N_PAD, D), jnp.float32)]

  @functools.partial(
      pl.kernel,
      out_type=out_type,
      mesh=mesh,
      scratch_types=[
          pltpu.VMEM((CPW, CE), jnp.int32),
          pltpu.VMEM((CPW, CE), jnp.int32),
          pltpu.VMEM((CE,), jnp.int32),
          pltpu.VMEM((CE, D), jnp.float32),
          pltpu.VMEM_SHARED((N_PAD, D), jnp.float32),
      ],
  )
  def seg_sum(y_hbm, src_hbm, dst_hbm, aux_hbm, *out_and_scratch):
    if with_counts:
      out_hbm, cnt_hbm, sidx, didx, dv, rows, acc = out_and_scratch
    else:
      out_hbm, sidx, didx, dv, rows, acc = out_and_scratch
      cnt_hbm = None
    c = lax.axis_index("c")
    s = lax.axis_index("s")
    wid = s * NC + c
    pltpu.sync_copy(aux_hbm.at[pl.ds(0, RPT)], acc.at[pl.ds(s * RPT, RPT)])
    pltpu.sync_copy(src_hbm.at[wid], sidx)
    pltpu.sync_copy(dst_hbm.at[wid], didx)
    plsc.subcore_barrier()

    @pl.loop(0, CPW)
    def _(j):
      for k16 in range(CE // 16):
        dv[pl.ds(k16 * 16, 16)] = didx[j, pl.ds(k16 * 16, 16)]
      pltpu.sync_copy(y_hbm.at[sidx.at[j]], rows)
      pltpu.sync_copy(rows, acc.at[dv], add=True)

    plsc.subcore_barrier()
    pltpu.sync_copy(acc.at[pl.ds(s * RPT, RPT)],
                    out_hbm.at[c, pl.ds(s * RPT, RPT)])

    if with_counts:
      plsc.subcore_barrier()
      pltpu.sync_copy(aux_hbm.at[pl.ds(0, RPT)], acc.at[pl.ds(s * RPT, RPT)])
      pltpu.sync_copy(aux_hbm.at[pl.ds(RPT, CE)], rows)
      plsc.subcore_barrier()

      @pl.loop(0, CPW)
      def _(j):
        for k16 in range(CE // 16):
          dv[pl.ds(k16 * 16, 16)] = didx[j, pl.ds(k16 * 16, 16)]
        pltpu.sync_copy(rows, acc.at[dv], add=True)

      plsc.subcore_barrier()
      pltpu.sync_copy(acc.at[pl.ds(s * RPT, RPT)],
                      cnt_hbm.at[c, pl.ds(s * RPT, RPT)])

  return seg_sum


_seg_sum_cnt = _make_seg_sum(True)
_seg_sum = _make_seg_sum(False)


_BLK = 2000
_GRID = N_NODES // _BLK


def _pre_body(x_ref, wl_ref, wr_ref, b_ref, y_ref, z_ref):
  xb = x_ref[...]
  y_ref[...] = lax.dot_general(xb, wl_ref[...], _DOT,
                               preferred_element_type=jnp.float32)
  z_ref[...] = lax.dot_general(xb, wr_ref[...], _DOT,
                               preferred_element_type=jnp.float32) + b_ref[...]


def _mid_body(p_ref, c_ref, z1_ref, wl_ref, wr_ref, b_ref,
              y2_ref, z2_ref, inv_ref):
  sacc = p_ref[0] + p_ref[1]
  cnt = c_ref[0, :, 0:1] + c_ref[1, :, 0:1]
  inv = 1.0 / jnp.maximum(cnt, 1.0)
  h = jnp.maximum(sacc * inv + z1_ref[...], 0.0)
  y2_ref[...] = lax.dot_general(h, wl_ref[...], _DOT,
                                preferred_element_type=jnp.float32)
  z2_ref[...] = lax.dot_general(h, wr_ref[...], _DOT,
                                preferred_element_type=jnp.float32) + b_ref[...]
  inv_ref[...] = jnp.broadcast_to(inv, (_BLK, D))


def _post_body(p_ref, z2_ref, inv_ref, o_ref):
  sacc = p_ref[0] + p_ref[1]
  o_ref[...] = sacc * inv_ref[...] + z2_ref[...]


def _row_spec():
  return pl.BlockSpec((_BLK, D), lambda i: (i, 0))


def _full_spec(shape):
  nd = len(shape)
  return pl.BlockSpec(shape, lambda i, _nd=nd: (0,) * _nd)


def _part_spec():
  return pl.BlockSpec((NC, _BLK, D), lambda i: (0, i, 0))


_pre = pl.pallas_call(
    _pre_body,
    grid=(_GRID,),
    in_specs=[_row_spec(), _full_spec((D, D)), _full_spec((D, D)),
              _full_spec((1, D))],
    out_specs=[_row_spec(), _row_spec()],
    out_shape=[jax.ShapeDtypeStruct((N_NODES, D), jnp.float32),
               jax.ShapeDtypeStruct((N_NODES, D), jnp.float32)],
)

_mid = pl.pallas_call(
    _mid_body,
    grid=(_GRID,),
    in_specs=[_part_spec(), _part_spec(), _row_spec(), _full_spec((D, D)),
              _full_spec((D, D)), _full_spec((1, D))],
    out_specs=[_row_spec(), _row_spec(), _row_spec()],
    out_shape=[jax.ShapeDtypeStruct((N_NODES, D), jnp.float32),
               jax.ShapeDtypeStruct((N_NODES, D), jnp.float32),
               jax.ShapeDtypeStruct((N_NODES, D), jnp.float32)],
)

_post = pl.pallas_call(
    _post_body,
    grid=(_GRID,),
    in_specs=[_part_spec(), _row_spec(), _row_spec()],
    out_specs=_row_spec(),
    out_shape=jax.ShapeDtypeStruct((N_NODES, D), jnp.float32),
)


def kernel(x, edge_index, W1l, b1l, W1r, W2l, b2l, W2r):
  src3 = edge_index[0].reshape(NW, CPW, CE)
  dst3 = edge_index[1].reshape(NW, CPW, CE)
  aux = jnp.concatenate([jnp.zeros((RPT, D), jnp.float32),
                         jnp.ones((CE, D), jnp.float32)], axis=0)

  y1, z1 = _pre(x, W1l, W1r, b1l.reshape(1, D))
  p1, pc = _seg_sum_cnt(y1, src3, dst3, aux)
  y2, z2, inv = _mid(p1, pc, z1, W2l, W2r, b2l.reshape(1, D))
  p2, = _seg_sum(y2, src3, dst3, aux)
  return _post(p2, z2, inv)


# speedup vs baseline: 10.1138x; 1.4773x over previous
import functools

import jax
import jax.numpy as jnp
from jax import lax
from jax.experimental import pallas as pl
from jax.experimental.pallas import tpu as pltpu
from jax.experimental.pallas import tpu_sc as plsc

N_NODES = 10000
D = 128
E_EDGES = 320000
CE = 80
NC = 2
NS = 16
NW = NC * NS
CPW = 126
E_PAD = NW * CPW * CE
N_PAD = 10240
RPT = N_PAD // NS

_NB = 2
_IS = 6
_NCS = 3

_DOT = (((1,), (1,)), ((), ()))


def _make_seg_sum(with_counts):
  mesh = plsc.VectorSubcoreMesh(core_axis_name="c", subcore_axis_name="s")
  out_type = [jax.ShapeDtypeStruct((NC, N_PAD, D), jnp.float32)]
  if with_counts:
    out_type = out_type + [jax.ShapeDtypeStruct((NC, N_PAD, D), jnp.float32)]
  n_out = 1 + int(with_counts)

  @functools.partial(
      pl.kernel,
      out_type=out_type,
      mesh=mesh,
      scratch_types=[
          pltpu.VMEM((CPW, CE), jnp.int32),
          pltpu.VMEM((_IS, CE), jnp.int32),
          pltpu.VMEM((_NB, CE, D), jnp.float32),
          pltpu.VMEM_SHARED((N_PAD, D), jnp.float32),
      ] + [pltpu.SemaphoreType.DMA] * (_IS + _NB + _NCS),
  )
  def seg_sum(y_hbm, src_hbm, dst_hbm, aux_hbm, *out_and_scratch):
    out_hbm = out_and_scratch[0]
    cnt_hbm = out_and_scratch[1] if with_counts else None
    didx, ivs, rows, acc = out_and_scratch[n_out:n_out + 4]
    sems = out_and_scratch[n_out + 4:]
    sis = sems[:_IS]
    sg = sems[_IS:_IS + _NB]
    sc_ = sems[_IS + _NB:]
    c = lax.axis_index("c")
    s = lax.axis_index("s")
    wid = s * NC + c
    base_e = wid * (CPW * CE)

    def src_sl(i):
      return src_hbm.at[pl.ds(base_e + i * CE, CE)]

    pltpu.sync_copy(aux_hbm.at[pl.ds(0, RPT)], acc.at[pl.ds(s * RPT, RPT)])
    pltpu.sync_copy(dst_hbm.at[wid], didx)
    plsc.subcore_barrier()

    for u in range(_IS):
      pltpu.async_copy(src_sl(u), ivs.at[u], sis[u])
    for u in range(_NB):
      pltpu.make_async_copy(src_sl(u), ivs.at[u], sis[u]).wait()
      pltpu.async_copy(y_hbm.at[ivs.at[u]], rows.at[u], sg[u])

    def row_slot(i, u, refill_idx, refill_gather):
      b = u % _NB
      pltpu.make_async_copy(y_hbm.at[ivs.at[u % _IS]], rows.at[b],
                            sg[b]).wait()
      pltpu.sync_copy(rows.at[b], acc.at[didx.at[i]], add=True)
      if refill_idx:
        pltpu.async_copy(src_sl(i + _IS), ivs.at[u % _IS], sis[u % _IS])
      if refill_gather:
        u2 = (u + _NB) % _IS
        pltpu.make_async_copy(src_sl(i + _NB), ivs.at[u2], sis[u2]).wait()
        pltpu.async_copy(y_hbm.at[ivs.at[u2]], rows.at[b], sg[b])

    @pl.loop(0, (CPW - _IS) // _IS)
    def _(k):
      for u in range(_IS):
        row_slot(k * _IS + u, u, True, True)

    for u in range(_IS):
      i = (CPW - _IS) + u
      row_slot(i, u, False, i + _NB < CPW)

    plsc.subcore_barrier()
    pltpu.sync_copy(acc.at[pl.ds(s * RPT, RPT)],
                    out_hbm.at[c, pl.ds(s * RPT, RPT)])

    if with_counts:
      plsc.subcore_barrier()
      pltpu.sync_copy(aux_hbm.at[pl.ds(0, RPT)], acc.at[pl.ds(s * RPT, RPT)])
      pltpu.sync_copy(aux_hbm.at[pl.ds(RPT, CE)], rows.at[0])
      plsc.subcore_barrier()
      ones = rows.at[0]

      def cnt_wait(i, u):
        pltpu.make_async_copy(ones, acc.at[didx.at[i]], sc_[u]).wait()

      for u in range(_NCS):
        pltpu.async_copy(ones, acc.at[didx.at[u]], sc_[u], add=True)

      @pl.loop(0, (CPW - _NCS) // _NCS)
      def _(k):
        for u in range(_NCS):
          i = k * _NCS + u
          cnt_wait(i, u)
          pltpu.async_copy(ones, acc.at[didx.at[i + _NCS]], sc_[u], add=True)

      for u in range(_NCS):
        cnt_wait((CPW - _NCS) + u, u)

      plsc.subcore_barrier()
      pltpu.sync_copy(acc.at[pl.ds(s * RPT, RPT)],
                      cnt_hbm.at[c, pl.ds(s * RPT, RPT)])

  return seg_sum


_seg_sum_cnt = _make_seg_sum(True)
_seg_sum = _make_seg_sum(False)


_BLK = 2000
_GRID = N_NODES // _BLK


def _pre_body(x_ref, wl_ref, wr_ref, b_ref, y_ref, z_ref):
  xb = x_ref[...]
  y_ref[...] = lax.dot_general(xb, wl_ref[...], _DOT,
                               preferred_element_type=jnp.float32)
  z_ref[...] = lax.dot_general(xb, wr_ref[...], _DOT,
                               preferred_element_type=jnp.float32) + b_ref[...]


def _mid_body(p_ref, c_ref, z1_ref, wl_ref, wr_ref, b_ref,
              y2_ref, z2_ref, inv_ref):
  sacc = p_ref[0] + p_ref[1]
  cnt = c_ref[0, :, 0:1] + c_ref[1, :, 0:1]
  inv = 1.0 / jnp.maximum(cnt, 1.0)
  h = jnp.maximum(sacc * inv + z1_ref[...], 0.0)
  y2_ref[...] = lax.dot_general(h, wl_ref[...], _DOT,
                                preferred_element_type=jnp.float32)
  z2_ref[...] = lax.dot_general(h, wr_ref[...], _DOT,
                                preferred_element_type=jnp.float32) + b_ref[...]
  inv_ref[...] = jnp.broadcast_to(inv, (_BLK, D))


def _post_body(p_ref, z2_ref, inv_ref, o_ref):
  sacc = p_ref[0] + p_ref[1]
  o_ref[...] = sacc * inv_ref[...] + z2_ref[...]


def _row_spec():
  return pl.BlockSpec((_BLK, D), lambda i: (i, 0))


def _full_spec(shape):
  nd = len(shape)
  return pl.BlockSpec(shape, lambda i, _nd=nd: (0,) * _nd)


def _part_spec():
  return pl.BlockSpec((NC, _BLK, D), lambda i: (0, i, 0))


_pre = pl.pallas_call(
    _pre_body,
    grid=(_GRID,),
    in_specs=[_row_spec(), _full_spec((D, D)), _full_spec((D, D)),
              _full_spec((1, D))],
    out_specs=[_row_spec(), _row_spec()],
    out_shape=[jax.ShapeDtypeStruct((N_NODES, D), jnp.float32),
               jax.ShapeDtypeStruct((N_NODES, D), jnp.float32)],
)

_mid = pl.pallas_call(
    _mid_body,
    grid=(_GRID,),
    in_specs=[_part_spec(), _part_spec(), _row_spec(), _full_spec((D, D)),
              _full_spec((D, D)), _full_spec((1, D))],
    out_specs=[_row_spec(), _row_spec(), _row_spec()],
    out_shape=[jax.ShapeDtypeStruct((N_NODES, D), jnp.float32),
               jax.ShapeDtypeStruct((N_NODES, D), jnp.float32),
               jax.ShapeDtypeStruct((N_NODES, D), jnp.float32)],
)

_post = pl.pallas_call(
    _post_body,
    grid=(_GRID,),
    in_specs=[_part_spec(), _row_spec(), _row_spec()],
    out_specs=_row_spec(),
    out_shape=jax.ShapeDtypeStruct((N_NODES, D), jnp.float32),
)


def kernel(x, edge_index, W1l, b1l, W1r, W2l, b2l, W2r):
  pad_n = E_PAD - E_EDGES
  pad_ar = jnp.arange(pad_n, dtype=jnp.int32)
  src1 = jnp.concatenate([edge_index[0], pad_ar % N_NODES])
  dst3 = jnp.concatenate(
      [edge_index[1], N_NODES + pad_ar % (N_PAD - N_NODES)]
  ).reshape(NW, CPW, CE)
  aux = jnp.concatenate([jnp.zeros((RPT, D), jnp.float32),
                         jnp.ones((CE, D), jnp.float32)], axis=0)

  y1, z1 = _pre(x, W1l, W1r, b1l.reshape(1, D))
  p1, pc = _seg_sum_cnt(y1, src1, dst3, aux)
  y2, z2, inv = _mid(p1, pc, z1, W2l, W2r, b2l.reshape(1, D))
  p2, = _seg_sum(y2, src1, dst3, aux)
  return _post(p2, z2, inv)


# speedup vs baseline: 12.2299x; 1.2092x over previous
import dataclasses
import functools

import jax
import jax.numpy as jnp
from jax import lax
from jax.experimental import pallas as pl
from jax.experimental.pallas import tpu as pltpu
from jax.experimental.pallas import tpu_sc as plsc

N_NODES = 10000
D = 128
E_EDGES = 320000
CE = 80
NC = 2
NS = 16
NW = NC * NS
CPW = 126
E_PAD = NW * CPW * CE
N_PAD = 10240
RPT = N_PAD // NS
CROWS = N_PAD // D

_NB = 2
_IS = 6

_DOT = (((1,), (1,)), ((), ()))

_SC_PARAMS = dataclasses.replace(
    pltpu.CompilerParams(), needs_layout_passes=False)


def _make_seg_sum(with_counts):
  mesh = plsc.VectorSubcoreMesh(core_axis_name="c", subcore_axis_name="s")
  out_type = [jax.ShapeDtypeStruct((NC, N_PAD, D), jnp.float32)]
  scratch = [
      pltpu.VMEM((_IS, CE), jnp.int32),
      pltpu.VMEM((_IS, CE), jnp.int32),
      pltpu.VMEM((_NB, CE, D), jnp.float32),
      pltpu.VMEM_SHARED((N_PAD, D), jnp.float32),
  ]
  if with_counts:
    out_type = out_type + [jax.ShapeDtypeStruct((NW, CROWS, D), jnp.float32)]
    scratch = scratch + [pltpu.VMEM((CROWS, D), jnp.float32)]
  n_out = 1 + int(with_counts)
  n_scr = len(scratch)

  @functools.partial(
      pl.kernel,
      out_type=out_type,
      mesh=mesh,
      scratch_types=scratch + [pltpu.SemaphoreType.DMA] * (2 * _IS + _NB),
      compiler_params=_SC_PARAMS,
  )
  def seg_sum(y_hbm, src_hbm, dst_hbm, aux_hbm, *rest):
    out_hbm = rest[0]
    cnt_hbm = rest[1] if with_counts else None
    ivs, ivd, rows, acc = rest[n_out:n_out + 4]
    cnt = rest[n_out + 4] if with_counts else None
    sems = rest[n_out + n_scr:]
    sis = sems[:_IS]
    sid = sems[_IS:2 * _IS]
    sg = sems[2 * _IS:]
    c = lax.axis_index("c")
    s = lax.axis_index("s")
    wid = s * NC + c
    base_e = wid * (CPW * CE)

    def src_sl(i):
      return src_hbm.at[pl.ds(base_e + i * CE, CE)]

    def dst_sl(i):
      return dst_hbm.at[pl.ds(base_e + i * CE, CE)]

    pltpu.sync_copy(aux_hbm.at[pl.ds(0, RPT)], acc.at[pl.ds(s * RPT, RPT)])
    if with_counts:
      pltpu.sync_copy(aux_hbm.at[pl.ds(0, CROWS)], cnt)
    plsc.subcore_barrier()

    for u in range(_IS):
      pltpu.async_copy(src_sl(u), ivs.at[u], sis[u])
      pltpu.async_copy(dst_sl(u), ivd.at[u], sid[u])
    for u in range(_NB):
      pltpu.make_async_copy(src_sl(u), ivs.at[u], sis[u]).wait()
      pltpu.async_copy(y_hbm.at[ivs.at[u]], rows.at[u], sg[u])

    ones16 = jnp.ones((16,), jnp.float32)

    def row_slot(i, u, refill, refill_g):
      b = u % _NB
      pltpu.make_async_copy(y_hbm.at[ivs.at[u]], rows.at[b], sg[b]).wait()
      pltpu.make_async_copy(dst_sl(i), ivd.at[u], sid[u]).wait()
      pltpu.sync_copy(rows.at[b], acc.at[ivd.at[u]], add=True)
      if with_counts:
        for k16 in range(CE // 16):
          v = ivd[u, pl.ds(k16 * 16, 16)]
          plsc.addupdate_scatter(
              cnt, [v >> 7, v & 127], ones16)
      if refill:
        pltpu.async_copy(src_sl(i + _IS), ivs.at[u], sis[u])
        pltpu.async_copy(dst_sl(i + _IS), ivd.at[u], sid[u])
      if refill_g:
        u2 = (u + _NB) % _IS
        pltpu.make_async_copy(src_sl(i + _NB), ivs.at[u2], sis[u2]).wait()
        pltpu.async_copy(y_hbm.at[ivs.at[u2]], rows.at[b], sg[b])

    @pl.loop(0, (CPW - _IS) // _IS)
    def _(k):
      for u in range(_IS):
        row_slot(k * _IS + u, u, True, True)

    for u in range(_IS):
      i = (CPW - _IS) + u
      row_slot(i, u, False, i + _NB < CPW)

    plsc.subcore_barrier()
    pltpu.sync_copy(acc.at[pl.ds(s * RPT, RPT)],
                    out_hbm.at[c, pl.ds(s * RPT, RPT)])
    if with_counts:
      pltpu.sync_copy(cnt, cnt_hbm.at[wid])

  return seg_sum


_seg_sum_cnt = _make_seg_sum(True)
_seg_sum = _make_seg_sum(False)


_BLK = 2048
_GRID = N_PAD // _BLK


def _pre_body(x_ref, wl_ref, wr_ref, b_ref, y_ref, z_ref):
  xb = x_ref[...]
  y_ref[...] = lax.dot_general(xb, wl_ref[...], _DOT,
                               preferred_element_type=jnp.float32)
  z_ref[...] = lax.dot_general(xb, wr_ref[...], _DOT,
                               preferred_element_type=jnp.float32) + b_ref[...]


def _mid_body(p_ref, c_ref, z1_ref, wl_ref, wr_ref, b_ref,
              y2_ref, z2_ref, inv_ref):
  sacc = p_ref[0] + p_ref[1]
  cnt = lax.dot_general(c_ref[...], jnp.ones((NW, 1), jnp.float32),
                        (((0,), (0,)), ((), ())),
                        preferred_element_type=jnp.float32)
  inv = 1.0 / jnp.maximum(cnt, 1.0)
  h = jnp.maximum(sacc * inv + z1_ref[...], 0.0)
  y2_ref[...] = lax.dot_general(h, wl_ref[...], _DOT,
                                preferred_element_type=jnp.float32)
  z2_ref[...] = lax.dot_general(h, wr_ref[...], _DOT,
                                preferred_element_type=jnp.float32) + b_ref[...]
  inv_ref[...] = jnp.broadcast_to(inv, (_BLK, D))


def _post_body(p_ref, z2_ref, inv_ref, o_ref):
  sacc = p_ref[0] + p_ref[1]
  o_ref[...] = sacc * inv_ref[...] + z2_ref[...]


def _row_spec():
  return pl.BlockSpec((_BLK, D), lambda i: (i, 0))


def _full_spec(shape):
  nd = len(shape)
  return pl.BlockSpec(shape, lambda i, _nd=nd: (0,) * _nd)


def _part_spec():
  return pl.BlockSpec((NC, _BLK, D), lambda i: (0, i, 0))


_pre = pl.pallas_call(
    _pre_body,
    grid=(_GRID,),
    in_specs=[_row_spec(), _full_spec((D, D)), _full_spec((D, D)),
              _full_spec((1, D))],
    out_specs=[_row_spec(), _row_spec()],
    out_shape=[jax.ShapeDtypeStruct((N_PAD, D), jnp.float32),
               jax.ShapeDtypeStruct((N_PAD, D), jnp.float32)],
)

_mid = pl.pallas_call(
    _mid_body,
    grid=(_GRID,),
    in_specs=[_part_spec(), pl.BlockSpec((NW, _BLK), lambda i: (0, i)),
              _row_spec(), _full_spec((D, D)), _full_spec((D, D)),
              _full_spec((1, D))],
    out_specs=[_row_spec(), _row_spec(), _row_spec()],
    out_shape=[jax.ShapeDtypeStruct((N_PAD, D), jnp.float32),
               jax.ShapeDtypeStruct((N_PAD, D), jnp.float32),
               jax.ShapeDtypeStruct((N_PAD, D), jnp.float32)],
)

_post = pl.pallas_call(
    _post_body,
    grid=(_GRID,),
    in_specs=[_part_spec(), _row_spec(), _row_spec()],
    out_specs=_row_spec(),
    out_shape=jax.ShapeDtypeStruct((N_PAD, D), jnp.float32),
)


def kernel(x, edge_index, W1l, b1l, W1r, W2l, b2l, W2r):
  pad_n = E_PAD - E_EDGES
  pad_ar = jnp.arange(pad_n, dtype=jnp.int32)
  src1 = jnp.concatenate([edge_index[0], pad_ar % N_NODES])
  dst1 = jnp.concatenate([edge_index[1],
                          N_NODES + pad_ar % (N_PAD - N_NODES)])
  x_pad = jnp.concatenate([x, jnp.zeros((N_PAD - N_NODES, D), x.dtype)])
  aux = jnp.zeros((RPT, D), jnp.float32)

  y1, z1 = _pre(x_pad, W1l, W1r, b1l.reshape(1, D))
  p1, pc = _seg_sum_cnt(y1, src1, dst1, aux)
  y2, z2, inv = _mid(p1, pc.reshape(NW, N_PAD), z1, W2l, W2r,
                     b2l.reshape(1, D))
  p2, = _seg_sum(y2, src1, dst1, aux)
  return _post(p2, z2, inv)[:N_NODES]


# speedup vs baseline: 13.1736x; 1.0772x over previous
import dataclasses
import functools

import jax
import jax.numpy as jnp
from jax import lax
from jax.experimental import pallas as pl
from jax.experimental.pallas import tpu as pltpu
from jax.experimental.pallas import tpu_sc as plsc

N_NODES = 10000
D = 128
E_EDGES = 320000
CE = 64
NC = 2
NS = 16
NW = NC * NS
CPW = 162
E_PAD = NW * CPW * CE
N_PAD = 10240
RPT = N_PAD // NS
CROWS = N_PAD // D

_NB = 3
_IS = 6
_SS = 2

_DOT = (((1,), (1,)), ((), ()))

_SC_PARAMS = dataclasses.replace(
    pltpu.CompilerParams(), needs_layout_passes=False)


def _make_seg_sum(with_counts):
  mesh = plsc.VectorSubcoreMesh(core_axis_name="c", subcore_axis_name="s")
  out_type = [jax.ShapeDtypeStruct((NC, N_PAD, D), jnp.float32)]
  scratch = [
      pltpu.VMEM((_IS, CE), jnp.int32),
      pltpu.VMEM((_IS, CE), jnp.int32),
      pltpu.VMEM((_NB, CE, D), jnp.float32),
      pltpu.VMEM_SHARED((N_PAD, D), jnp.float32),
  ]
  if with_counts:
    out_type = out_type + [jax.ShapeDtypeStruct((NW, CROWS, D), jnp.float32)]
    scratch = scratch + [pltpu.VMEM((CROWS, D), jnp.float32)]
  n_out = 1 + int(with_counts)
  n_scr = len(scratch)

  @functools.partial(
      pl.kernel,
      out_type=out_type,
      mesh=mesh,
      scratch_types=scratch + [pltpu.SemaphoreType.DMA] * (2 * _IS + _NB + _SS),
      compiler_params=_SC_PARAMS,
  )
  def seg_sum(y_hbm, src_hbm, dst_hbm, aux_hbm, *rest):
    out_hbm = rest[0]
    cnt_hbm = rest[1] if with_counts else None
    ivs, ivd, rows, acc = rest[n_out:n_out + 4]
    cnt = rest[n_out + 4] if with_counts else None
    sems = rest[n_out + n_scr:]
    sis = sems[:_IS]
    sid = sems[_IS:2 * _IS]
    sg = sems[2 * _IS:2 * _IS + _NB]
    ss = sems[2 * _IS + _NB:]
    c = lax.axis_index("c")
    s = lax.axis_index("s")
    wid = s * NC + c
    base_e = wid * (CPW * CE)

    def src_sl(i):
      return src_hbm.at[pl.ds(base_e + i * CE, CE)]

    def dst_sl(i):
      return dst_hbm.at[pl.ds(base_e + i * CE, CE)]

    def scat(u):
      return pltpu.make_async_copy(rows.at[u % _NB], acc.at[ivd.at[u % _IS]],
                                   ss[u % _SS])

    pltpu.sync_copy(aux_hbm.at[pl.ds(0, RPT)], acc.at[pl.ds(s * RPT, RPT)])
    if with_counts:
      pltpu.sync_copy(aux_hbm.at[pl.ds(0, CROWS)], cnt)
    plsc.subcore_barrier()

    for u in range(_IS):
      pltpu.async_copy(src_sl(u), ivs.at[u], sis[u])
    for u in range(_IS - 1):
      pltpu.async_copy(dst_sl(u), ivd.at[u], sid[u])
    for u in range(2):
      pltpu.make_async_copy(src_sl(u), ivs.at[u], sis[u]).wait()
      pltpu.async_copy(y_hbm.at[ivs.at[u]], rows.at[u], sg[u])

    ones16 = jnp.ones((16,), jnp.float32)

    def row_slot(i, u, first, refill_s, refill_d, refill_g):
      pltpu.make_async_copy(y_hbm.at[ivs.at[u]], rows.at[u % _NB],
                            sg[u % _NB]).wait()
      pltpu.make_async_copy(dst_sl(i), ivd.at[u], sid[u]).wait()
      scat(u).start(add=True)
      if with_counts:
        for k16 in range(CE // 16):
          v = ivd[u, pl.ds(k16 * 16, 16)]
          plsc.addupdate_scatter(cnt, [v >> 7, v & 127], ones16)
      if refill_s:
        pltpu.async_copy(src_sl(i + _IS), ivs.at[u], sis[u])
      if not first:
        scat(u + _IS - 1).wait()
      if refill_d:
        u5 = (u + _IS - 1) % _IS
        pltpu.async_copy(dst_sl(i + _IS - 1), ivd.at[u5], sid[u5])
      if refill_g:
        u2 = (u + 2) % _IS
        pltpu.make_async_copy(src_sl(i + 2), ivs.at[u2], sis[u2]).wait()
        pltpu.async_copy(y_hbm.at[ivs.at[u2]], rows.at[(u + 2) % _NB],
                         sg[(u + 2) % _NB])

    for u in range(_IS):
      row_slot(u, u, u == 0, True, True, True)

    @pl.loop(1, CPW // _IS - 1)
    def _(k):
      for u in range(_IS):
        row_slot(k * _IS + u, u, False, True, True, True)

    for u in range(_IS):
      i = (CPW - _IS) + u
      row_slot(i, u, False, i + _IS < CPW, i + _IS - 1 < CPW, i + 2 < CPW)
    scat(CPW - 1).wait()

    plsc.subcore_barrier()
    pltpu.sync_copy(acc.at[pl.ds(s * RPT, RPT)],
                    out_hbm.at[c, pl.ds(s * RPT, RPT)])
    if with_counts:
      pltpu.sync_copy(cnt, cnt_hbm.at[wid])

  return seg_sum


_seg_sum_cnt = _make_seg_sum(True)
_seg_sum = _make_seg_sum(False)


_BLK = 2048
_GRID = N_PAD // _BLK


def _pre_body(x_ref, wl_ref, wr_ref, b_ref, y_ref, z_ref):
  xb = x_ref[...]
  y_ref[...] = lax.dot_general(xb, wl_ref[...], _DOT,
                               preferred_element_type=jnp.float32)
  z_ref[...] = lax.dot_general(xb, wr_ref[...], _DOT,
                               preferred_element_type=jnp.float32) + b_ref[...]


def _mid_body(p_ref, c_ref, z1_ref, wl_ref, wr_ref, b_ref,
              y2_ref, z2_ref, inv_ref):
  sacc = p_ref[0] + p_ref[1]
  cnt = lax.dot_general(c_ref[...], jnp.ones((NW, 1), jnp.float32),
                        (((0,), (0,)), ((), ())),
                        preferred_element_type=jnp.float32)
  inv = 1.0 / jnp.maximum(cnt, 1.0)
  h = jnp.maximum(sacc * inv + z1_ref[...], 0.0)
  y2_ref[...] = lax.dot_general(h, wl_ref[...], _DOT,
                                preferred_element_type=jnp.float32)
  z2_ref[...] = lax.dot_general(h, wr_ref[...], _DOT,
                                preferred_element_type=jnp.float32) + b_ref[...]
  inv_ref[...] = jnp.broadcast_to(inv, (_BLK, D))


def _post_body(p_ref, z2_ref, inv_ref, o_ref):
  sacc = p_ref[0] + p_ref[1]
  o_ref[...] = sacc * inv_ref[...] + z2_ref[...]


def _row_spec():
  return pl.BlockSpec((_BLK, D), lambda i: (i, 0))


def _full_spec(shape):
  nd = len(shape)
  return pl.BlockSpec(shape, lambda i, _nd=nd: (0,) * _nd)


def _part_spec():
  return pl.BlockSpec((NC, _BLK, D), lambda i: (0, i, 0))


_pre = pl.pallas_call(
    _pre_body,
    grid=(_GRID,),
    in_specs=[_row_spec(), _full_spec((D, D)), _full_spec((D, D)),
              _full_spec((1, D))],
    out_specs=[_row_spec(), _row_spec()],
    out_shape=[jax.ShapeDtypeStruct((N_PAD, D), jnp.float32),
               jax.ShapeDtypeStruct((N_PAD, D), jnp.float32)],
)

_mid = pl.pallas_call(
    _mid_body,
    grid=(_GRID,),
    in_specs=[_part_spec(), pl.BlockSpec((NW, _BLK), lambda i: (0, i)),
              _row_spec(), _full_spec((D, D)), _full_spec((D, D)),
              _full_spec((1, D))],
    out_specs=[_row_spec(), _row_spec(), _row_spec()],
    out_shape=[jax.ShapeDtypeStruct((N_PAD, D), jnp.float32),
               jax.ShapeDtypeStruct((N_PAD, D), jnp.float32),
               jax.ShapeDtypeStruct((N_PAD, D), jnp.float32)],
)

_post = pl.pallas_call(
    _post_body,
    grid=(_GRID,),
    in_specs=[_part_spec(), _row_spec(), _row_spec()],
    out_specs=_row_spec(),
    out_shape=jax.ShapeDtypeStruct((N_NODES, D), jnp.float32),
)


def kernel(x, edge_index, W1l, b1l, W1r, W2l, b2l, W2r):
  pad_n = E_PAD - E_EDGES
  pad_ar = jnp.arange(pad_n, dtype=jnp.int32)
  src1 = jnp.concatenate([edge_index[0], pad_ar % N_NODES])
  dst1 = jnp.concatenate([edge_index[1],
                          N_NODES + pad_ar % (N_PAD - N_NODES)])
  x_pad = jnp.concatenate([x, jnp.zeros((N_PAD - N_NODES, D), x.dtype)])
  aux = jnp.zeros((RPT, D), jnp.float32)

  y1, z1 = _pre(x_pad, W1l, W1r, b1l.reshape(1, D))
  p1, pc = _seg_sum_cnt(y1, src1, dst1, aux)
  y2, z2, inv = _mid(p1, pc.reshape(NW, N_PAD), z1, W2l, W2r,
                     b2l.reshape(1, D))
  p2, = _seg_sum(y2, src1, dst1, aux)
  return _post(p2, z2, inv)


# speedup vs baseline: 13.2770x; 1.0078x over previous
import dataclasses
import functools

import jax
import jax.numpy as jnp
from jax import lax
from jax.experimental import pallas as pl
from jax.experimental.pallas import tpu as pltpu
from jax.experimental.pallas import tpu_sc as plsc

N_NODES = 10000
D = 128
E_EDGES = 320000
NC = 2
NS = 16
NW = NC * NS
CE1, CPW1 = 64, 162
CE2, CPW2 = 80, 126
N_PAD = 10240
RPT = N_PAD // NS
CROWS = N_PAD // D

_NB = 3
_IS = 6
_SS = 2

_DOT = (((1,), (1,)), ((), ()))

_SC_PARAMS = dataclasses.replace(
    pltpu.CompilerParams(), needs_layout_passes=False)


def _make_seg_sum(with_counts, CE, CPW):
  mesh = plsc.VectorSubcoreMesh(core_axis_name="c", subcore_axis_name="s")
  out_type = [jax.ShapeDtypeStruct((NC, N_PAD, D), jnp.float32)]
  scratch = [
      pltpu.VMEM((_IS, CE), jnp.int32),
      pltpu.VMEM((_IS, CE), jnp.int32),
      pltpu.VMEM((_NB, CE, D), jnp.float32),
      pltpu.VMEM_SHARED((N_PAD, D), jnp.float32),
  ]
  if with_counts:
    out_type = out_type + [jax.ShapeDtypeStruct((NW, CROWS, D), jnp.float32)]
    scratch = scratch + [pltpu.VMEM((CROWS, D), jnp.float32)]
  n_out = 1 + int(with_counts)
  n_scr = len(scratch)

  @functools.partial(
      pl.kernel,
      out_type=out_type,
      mesh=mesh,
      scratch_types=scratch + [pltpu.SemaphoreType.DMA] * (2 * _IS + _NB + _SS),
      compiler_params=_SC_PARAMS,
  )
  def seg_sum(y_hbm, src_hbm, dst_hbm, aux_hbm, *rest):
    out_hbm = rest[0]
    cnt_hbm = rest[1] if with_counts else None
    ivs, ivd, rows, acc = rest[n_out:n_out + 4]
    cnt = rest[n_out + 4] if with_counts else None
    sems = rest[n_out + n_scr:]
    sis = sems[:_IS]
    sid = sems[_IS:2 * _IS]
    sg = sems[2 * _IS:2 * _IS + _NB]
    ss = sems[2 * _IS + _NB:]
    c = lax.axis_index("c")
    s = lax.axis_index("s")
    wid = s * NC + c
    base_e = wid * (CPW * CE)

    def src_sl(i):
      return src_hbm.at[pl.ds(base_e + i * CE, CE)]

    def dst_sl(i):
      return dst_hbm.at[pl.ds(base_e + i * CE, CE)]

    def scat(u):
      return pltpu.make_async_copy(rows.at[u % _NB], acc.at[ivd.at[u % _IS]],
                                   ss[u % _SS])

    pltpu.sync_copy(aux_hbm.at[pl.ds(0, RPT)], acc.at[pl.ds(s * RPT, RPT)])
    if with_counts:
      pltpu.sync_copy(aux_hbm.at[pl.ds(0, CROWS)], cnt)
    plsc.subcore_barrier()

    for u in range(_IS):
      pltpu.async_copy(src_sl(u), ivs.at[u], sis[u])
    for u in range(_IS - 1):
      pltpu.async_copy(dst_sl(u), ivd.at[u], sid[u])
    for u in range(2):
      pltpu.make_async_copy(src_sl(u), ivs.at[u], sis[u]).wait()
      pltpu.async_copy(y_hbm.at[ivs.at[u]], rows.at[u], sg[u])

    ones16 = jnp.ones((16,), jnp.float32)

    def row_slot(i, u, first, refill_s, refill_d, refill_g):
      pltpu.make_async_copy(y_hbm.at[ivs.at[u]], rows.at[u % _NB],
                            sg[u % _NB]).wait()
      pltpu.make_async_copy(dst_sl(i), ivd.at[u], sid[u]).wait()
      scat(u).start(add=True)
      if with_counts:
        for k16 in range(CE // 16):
          v = ivd[u, pl.ds(k16 * 16, 16)]
          plsc.addupdate_scatter(cnt, [v >> 7, v & 127], ones16)
      if refill_s:
        pltpu.async_copy(src_sl(i + _IS), ivs.at[u], sis[u])
      if not first:
        scat(u + _IS - 1).wait()
      if refill_d:
        u5 = (u + _IS - 1) % _IS
        pltpu.async_copy(dst_sl(i + _IS - 1), ivd.at[u5], sid[u5])
      if refill_g:
        u2 = (u + 2) % _IS
        pltpu.make_async_copy(src_sl(i + 2), ivs.at[u2], sis[u2]).wait()
        pltpu.async_copy(y_hbm.at[ivs.at[u2]], rows.at[(u + 2) % _NB],
                         sg[(u + 2) % _NB])

    for u in range(_IS):
      row_slot(u, u, u == 0, True, True, True)

    @pl.loop(1, CPW // _IS - 1)
    def _(k):
      for u in range(_IS):
        row_slot(k * _IS + u, u, False, True, True, True)

    for u in range(_IS):
      i = (CPW - _IS) + u
      row_slot(i, u, False, i + _IS < CPW, i + _IS - 1 < CPW, i + 2 < CPW)
    scat(CPW - 1).wait()

    plsc.subcore_barrier()
    pltpu.sync_copy(acc.at[pl.ds(s * RPT, RPT)],
                    out_hbm.at[c, pl.ds(s * RPT, RPT)])
    if with_counts:
      pltpu.sync_copy(cnt, cnt_hbm.at[wid])

  return seg_sum


_seg_sum_cnt = _make_seg_sum(True, CE1, CPW1)
_seg_sum = _make_seg_sum(False, CE2, CPW2)


_BLK = 2048
_GRID = N_PAD // _BLK


def _matmul_body(x_ref, w_ref, y_ref):
  y_ref[...] = lax.dot_general(x_ref[...], w_ref[...], _DOT,
                               preferred_element_type=jnp.float32)


def _matmul_bias_body(x_ref, w_ref, b_ref, y_ref):
  y_ref[...] = lax.dot_general(x_ref[...], w_ref[...], _DOT,
                               preferred_element_type=jnp.float32) + b_ref[...]


def _mid_body(p_ref, c_ref, z1_ref, wl_ref, y2_ref, h_ref, inv_ref):
  sacc = p_ref[0] + p_ref[1]
  cnt = lax.dot_general(c_ref[...], jnp.ones((NW, 1), jnp.float32),
                        (((0,), (0,)), ((), ())),
                        preferred_element_type=jnp.float32)
  inv = 1.0 / jnp.maximum(cnt, 1.0)
  h = jnp.maximum(sacc * inv + z1_ref[...], 0.0)
  y2_ref[...] = lax.dot_general(h, wl_ref[...], _DOT,
                                preferred_element_type=jnp.float32)
  h_ref[...] = h
  inv_ref[...] = jnp.broadcast_to(inv, (_BLK, D))


def _post_body(p_ref, z2_ref, inv_ref, o_ref):
  sacc = p_ref[0] + p_ref[1]
  o_ref[...] = sacc * inv_ref[...] + z2_ref[...]


def _row_spec():
  return pl.BlockSpec((_BLK, D), lambda i: (i, 0))


def _full_spec(shape):
  nd = len(shape)
  return pl.BlockSpec(shape, lambda i, _nd=nd: (0,) * _nd)


def _part_spec():
  return pl.BlockSpec((NC, _BLK, D), lambda i: (0, i, 0))


_row_f32 = jax.ShapeDtypeStruct((N_PAD, D), jnp.float32)

_matmul = pl.pallas_call(
    _matmul_body,
    grid=(_GRID,),
    in_specs=[_row_spec(), _full_spec((D, D))],
    out_specs=_row_spec(),
    out_shape=_row_f32,
)

_matmul_bias = pl.pallas_call(
    _matmul_bias_body,
    grid=(_GRID,),
    in_specs=[_row_spec(), _full_spec((D, D)), _full_spec((1, D))],
    out_specs=_row_spec(),
    out_shape=_row_f32,
)

_mid = pl.pallas_call(
    _mid_body,
    grid=(_GRID,),
    in_specs=[_part_spec(), pl.BlockSpec((NW, _BLK), lambda i: (0, i)),
              _row_spec(), _full_spec((D, D))],
    out_specs=[_row_spec(), _row_spec(), _row_spec()],
    out_shape=[_row_f32, _row_f32, _row_f32],
)

_post = pl.pallas_call(
    _post_body,
    grid=(_GRID,),
    in_specs=[_part_spec(), _row_spec(), _row_spec()],
    out_specs=_row_spec(),
    out_shape=jax.ShapeDtypeStruct((N_NODES, D), jnp.float32),
)


def _pad_edges(edge_index, e_pad):
  pad_ar = jnp.arange(e_pad - E_EDGES, dtype=jnp.int32)
  src1 = jnp.concatenate([edge_index[0], pad_ar % N_NODES])
  dst1 = jnp.concatenate([edge_index[1],
                          N_NODES + pad_ar % (N_PAD - N_NODES)])
  return src1, dst1


def kernel(x, edge_index, W1l, b1l, W1r, W2l, b2l, W2r):
  src1a, dst1a = _pad_edges(edge_index, NW * CPW1 * CE1)
  src1b, dst1b = _pad_edges(edge_index, NW * CPW2 * CE2)
  x_pad = jnp.concatenate([x, jnp.zeros((N_PAD - N_NODES, D), x.dtype)])
  aux = jnp.zeros((RPT, D), jnp.float32)

  y1 = _matmul(x_pad, W1l)
  z1 = _matmul_bias(x_pad, W1r, b1l.reshape(1, D))
  p1, pc = _seg_sum_cnt(y1, src1a, dst1a, aux)
  y2, h, inv = _mid(p1, pc.reshape(NW, N_PAD), z1, W2l)
  z2 = _matmul_bias(h, W2r, b2l.reshape(1, D))
  p2, = _seg_sum(y2, src1b, dst1b, aux)
  return _post(p2, z2, inv)


# speedup vs baseline: 14.0326x; 1.0569x over previous
import dataclasses
import functools

import jax
import jax.numpy as jnp
from jax import lax
from jax.experimental import pallas as pl
from jax.experimental.pallas import tpu as pltpu
from jax.experimental.pallas import tpu_sc as plsc

N_NODES = 10000
D = 128
E_EDGES = 320000
NC = 2
NS = 16
NW = NC * NS
CE1, CPW1 = 64, 162
CE2, CPW2 = 80, 126
N_PAD = 10240
RPT = N_PAD // NS
CROWS = N_PAD // D

_NB = 3
_IS = 6
_SS = 2

_DOT = (((1,), (1,)), ((), ()))

_SC_PARAMS = dataclasses.replace(
    pltpu.CompilerParams(), needs_layout_passes=False)


def _make_seg_sum(with_counts, CE, CPW):
  mesh = plsc.VectorSubcoreMesh(core_axis_name="c", subcore_axis_name="s")
  out_type = [jax.ShapeDtypeStruct((NC, N_PAD, D), jnp.float32)]
  scratch = [
      pltpu.VMEM((_IS, CE), jnp.int32),
      pltpu.VMEM((_IS, CE), jnp.int32),
      pltpu.VMEM((_NB, CE, D), jnp.float32),
      pltpu.VMEM_SHARED((N_PAD, D), jnp.float32),
  ]
  if with_counts:
    out_type = out_type + [jax.ShapeDtypeStruct((NW, CROWS, D), jnp.float32)]
    scratch = scratch + [pltpu.VMEM((CROWS, D), jnp.float32)]
  n_out = 1 + int(with_counts)
  n_scr = len(scratch)

  @functools.partial(
      pl.kernel,
      out_type=out_type,
      mesh=mesh,
      scratch_types=scratch + [pltpu.SemaphoreType.DMA] * (2 * _IS + _NB + _SS),
      compiler_params=_SC_PARAMS,
  )
  def seg_sum(y_hbm, src_hbm, dst_hbm, *rest):
    out_hbm = rest[0]
    cnt_hbm = rest[1] if with_counts else None
    ivs, ivd, rows, acc = rest[n_out:n_out + 4]
    cnt = rest[n_out + 4] if with_counts else None
    sems = rest[n_out + n_scr:]
    sis = sems[:_IS]
    sid = sems[_IS:2 * _IS]
    sg = sems[2 * _IS:2 * _IS + _NB]
    ss = sems[2 * _IS + _NB:]
    c = lax.axis_index("c")
    s = lax.axis_index("s")
    wid = s * NC + c
    base_e = wid * (CPW * CE)

    def src_sl(i):
      return src_hbm.at[pl.ds(base_e + i * CE, CE)]

    def dst_sl(i):
      return dst_hbm.at[pl.ds(base_e + i * CE, CE)]

    def scat(u):
      return pltpu.make_async_copy(rows.at[u % _NB], acc.at[ivd.at[u % _IS]],
                                   ss[u % _SS])

    z16 = jnp.zeros((16,), jnp.float32)

    @pl.loop(0, CE)
    def _(r):
      for k16 in range(D // 16):
        rows[0, r, pl.ds(k16 * 16, 16)] = z16

    @pl.loop(0, RPT // CE)
    def _(j):
      pltpu.sync_copy(rows.at[0], acc.at[pl.ds(s * RPT + j * CE, CE)])

    if with_counts:
      @pl.loop(0, CROWS)
      def _(r):
        for k16 in range(D // 16):
          cnt[r, pl.ds(k16 * 16, 16)] = z16

    plsc.subcore_barrier()

    for u in range(_IS):
      pltpu.async_copy(src_sl(u), ivs.at[u], sis[u])
    for u in range(_IS - 1):
      pltpu.async_copy(dst_sl(u), ivd.at[u], sid[u])
    for u in range(2):
      pltpu.make_async_copy(src_sl(u), ivs.at[u], sis[u]).wait()
      pltpu.async_copy(y_hbm.at[ivs.at[u]], rows.at[u], sg[u])

    ones16 = jnp.ones((16,), jnp.float32)

    def row_slot(i, u, first, refill_s, refill_d, refill_g):
      pltpu.make_async_copy(y_hbm.at[ivs.at[u]], rows.at[u % _NB],
                            sg[u % _NB]).wait()
      pltpu.make_async_copy(dst_sl(i), ivd.at[u], sid[u]).wait()
      scat(u).start(add=True)
      if with_counts:
        for k16 in range(CE // 16):
          v = ivd[u, pl.ds(k16 * 16, 16)]
          plsc.addupdate_scatter(cnt, [v >> 7, v & 127], ones16)
      if refill_s:
        pltpu.async_copy(src_sl(i + _IS), ivs.at[u], sis[u])
      if not first:
        scat(u + _IS - 1).wait()
      if refill_d:
        u5 = (u + _IS - 1) % _IS
        pltpu.async_copy(dst_sl(i + _IS - 1), ivd.at[u5], sid[u5])
      if refill_g:
        u2 = (u + 2) % _IS
        pltpu.make_async_copy(src_sl(i + 2), ivs.at[u2], sis[u2]).wait()
        pltpu.async_copy(y_hbm.at[ivs.at[u2]], rows.at[(u + 2) % _NB],
                         sg[(u + 2) % _NB])

    for u in range(_IS):
      row_slot(u, u, u == 0, True, True, True)

    @pl.loop(1, CPW // _IS - 1)
    def _(k):
      for u in range(_IS):
        row_slot(k * _IS + u, u, False, True, True, True)

    for u in range(_IS):
      i = (CPW - _IS) + u
      row_slot(i, u, False, i + _IS < CPW, i + _IS - 1 < CPW, i + 2 < CPW)
    scat(CPW - 1).wait()

    plsc.subcore_barrier()
    pltpu.sync_copy(acc.at[pl.ds(s * RPT, RPT)],
                    out_hbm.at[c, pl.ds(s * RPT, RPT)])
    if with_counts:
      pltpu.sync_copy(cnt, cnt_hbm.at[wid])

  return seg_sum


_seg_sum_cnt = _make_seg_sum(True, CE1, CPW1)
_seg_sum = _make_seg_sum(False, CE2, CPW2)


_BLK = 2048
_GRID = N_PAD // _BLK


def _matmul_body(x_ref, w_ref, y_ref):
  y_ref[...] = lax.dot_general(x_ref[...], w_ref[...], _DOT,
                               preferred_element_type=jnp.float32)


def _matmul_bias_body(x_ref, w_ref, b_ref, y_ref):
  y_ref[...] = lax.dot_general(x_ref[...], w_ref[...], _DOT,
                               preferred_element_type=jnp.float32) + b_ref[...]


def _mid_body(p_ref, c_ref, z1_ref, wl_ref, y2_ref, h_ref, inv_ref):
  sacc = p_ref[0] + p_ref[1]
  cnt = lax.dot_general(c_ref[...], jnp.ones((NW, 1), jnp.float32),
                        (((0,), (0,)), ((), ())),
                        preferred_element_type=jnp.float32)
  inv = 1.0 / jnp.maximum(cnt, 1.0)
  h = jnp.maximum(sacc * inv + z1_ref[...], 0.0)
  y2_ref[...] = lax.dot_general(h, wl_ref[...], _DOT,
                                preferred_element_type=jnp.float32)
  h_ref[...] = h
  inv_ref[...] = jnp.broadcast_to(inv, (_BLK, D))


def _post_body(p_ref, z2_ref, inv_ref, o_ref):
  sacc = p_ref[0] + p_ref[1]
  o_ref[...] = sacc * inv_ref[...] + z2_ref[...]


def _row_spec():
  return pl.BlockSpec((_BLK, D), lambda i: (i, 0))


def _full_spec(shape):
  nd = len(shape)
  return pl.BlockSpec(shape, lambda i, _nd=nd: (0,) * _nd)


def _part_spec():
  return pl.BlockSpec((NC, _BLK, D), lambda i: (0, i, 0))


_row_f32 = jax.ShapeDtypeStruct((N_PAD, D), jnp.float32)

_matmul = pl.pallas_call(
    _matmul_body,
    grid=(_GRID,),
    in_specs=[_row_spec(), _full_spec((D, D))],
    out_specs=_row_spec(),
    out_shape=_row_f32,
)

_matmul_bias = pl.pallas_call(
    _matmul_bias_body,
    grid=(_GRID,),
    in_specs=[_row_spec(), _full_spec((D, D)), _full_spec((1, D))],
    out_specs=_row_spec(),
    out_shape=_row_f32,
)

_mid = pl.pallas_call(
    _mid_body,
    grid=(_GRID,),
    in_specs=[_part_spec(), pl.BlockSpec((NW, _BLK), lambda i: (0, i)),
              _row_spec(), _full_spec((D, D))],
    out_specs=[_row_spec(), _row_spec(), _row_spec()],
    out_shape=[_row_f32, _row_f32, _row_f32],
)

_post = pl.pallas_call(
    _post_body,
    grid=(_GRID,),
    in_specs=[_part_spec(), _row_spec(), _row_spec()],
    out_specs=_row_spec(),
    out_shape=jax.ShapeDtypeStruct((N_NODES, D), jnp.float32),
)


def _pad_edges(edge_index, e_pad):
  pad_ar = jnp.arange(e_pad - E_EDGES, dtype=jnp.int32)
  src1 = jnp.concatenate([edge_index[0], pad_ar % N_NODES])
  dst1 = jnp.concatenate([edge_index[1],
                          N_NODES + pad_ar % (N_PAD - N_NODES)])
  return src1, dst1


def kernel(x, edge_index, W1l, b1l, W1r, W2l, b2l, W2r):
  src1a, dst1a = _pad_edges(edge_index, NW * CPW1 * CE1)
  src1b, dst1b = _pad_edges(edge_index, NW * CPW2 * CE2)
  x_pad = jnp.concatenate([x, jnp.zeros((N_PAD - N_NODES, D), x.dtype)])

  y1 = _matmul(x_pad, W1l)
  z1 = _matmul_bias(x_pad, W1r, b1l.reshape(1, D))
  p1, pc = _seg_sum_cnt(y1, src1a, dst1a)
  y2, h, inv = _mid(p1, pc.reshape(NW, N_PAD), z1, W2l)
  z2 = _matmul_bias(h, W2r, b2l.reshape(1, D))
  p2, = _seg_sum(y2, src1b, dst1b)
  return _post(p2, z2, inv)
